# baseline (speedup 1.0000x reference)
import functools

import jax
import jax.numpy as jnp
from jax.experimental import pallas as pl

_N_LAYERS = 2
_BLOCK_ROWS = 256


def _layer_mean_kernel(u_ref, i_ref, uo_ref, io_ref):
    inv = 1.0 / _N_LAYERS
    u_acc = u_ref[:]
    i_acc = i_ref[:]
    for _ in range(_N_LAYERS - 1):
        u_acc = u_acc + u_ref[:]
        i_acc = i_acc + i_ref[:]
    uo_ref[:] = u_acc * inv
    io_ref[:] = i_acc * inv


@functools.partial(jax.jit, static_argnums=())
def kernel(user_emb, item_emb, adj_rows, adj_cols, norm_vals, adj_vals,
           w_q, b_q, w_k, b_k, w_v, b_v):
    n_user, emb = user_emb.shape
    n_item, _ = item_emb.shape
    grid = (n_user // _BLOCK_ROWS,)
    spec = pl.BlockSpec((_BLOCK_ROWS, emb), lambda r: (r, 0))
    user_out, item_out = pl.pallas_call(
        _layer_mean_kernel,
        grid=grid,
        in_specs=[spec, spec],
        out_specs=[spec, spec],
        out_shape=[
            jax.ShapeDtypeStruct((n_user, emb), user_emb.dtype),
            jax.ShapeDtypeStruct((n_item, emb), item_emb.dtype),
        ],
    )(user_emb, item_emb)
    return user_out, item_out
